# baseline (speedup 1.0000x reference)
import jax
import jax.numpy as jnp
from jax import lax
from jax.experimental import pallas as pl
from jax.experimental.pallas import tpu as pltpu
from jax.experimental.pallas import tpu_sc as plsc

B = 1024
K = 100000
L = 16
NW = 32
RPW = B // NW
G = 8
NG = RPW // G
KT = 2048
NFULL = 48
NVK = KT // L
KREM_OFF = NFULL * KT
KREM = 1664
KMAIN = KREM_OFF + KREM
KTAIL = K - KMAIN
NSL = 4
NJ = NFULL // NSL


def _sc_body(s_hbm, v_hbm, o_hbm, sbuf, vbuf, rbuf, vrbuf, tbuf, obuf, tmat,
             sems):
    cid = lax.axis_index("c")
    sid = lax.axis_index("s")
    wid = sid * 2 + cid
    base = wid * RPW
    lane = lax.iota(jnp.int32, L)

    for g in range(NG):
        row0 = base + g * G

        def sdma(ck, slot):
            return pltpu.make_async_copy(
                s_hbm.at[pl.ds(row0, G), pl.ds(ck * KT, KT)],
                sbuf.at[slot], sems.at[slot])

        def vdma(ck, slot):
            return pltpu.make_async_copy(
                v_hbm.at[pl.ds(ck * KT, KT)], vbuf.at[slot], sems.at[slot])

        rcp = pltpu.make_async_copy(
            s_hbm.at[pl.ds(row0, G), pl.ds(KREM_OFF, KREM)], rbuf,
            sems.at[NSL])
        tcp = pltpu.make_async_copy(
            s_hbm.at[pl.ds(row0, G), pl.ds(KMAIN, KTAIL)], tbuf,
            sems.at[NSL + 1])
        vrcp = pltpu.make_async_copy(
            v_hbm.at[pl.ds(KREM_OFF, KREM + KTAIL)], vrbuf, sems.at[NSL])
        rcp.start()
        tcp.start()
        vrcp.start()
        for ck in range(NSL - 1):
            sdma(ck, ck).start()
            vdma(ck, ck).start()

        def compute_chunk(accs, slot, nv):
            def vstep(i, accs):
                off = i * L
                v16 = vbuf[slot, pl.ds(off, L)]
                return tuple(
                    accs[r] + sbuf[slot, r, pl.ds(off, L)] * v16
                    for r in range(G))

            return lax.fori_loop(0, nv, vstep, accs)

        def jbody(j, accs):
            for slot in range(NSL):
                ck = NSL * j + slot
                sdma(ck, slot).wait()
                vdma(ck, slot).wait()
                accs = compute_chunk(accs, slot, NVK)

                nslot = (slot + NSL - 1) % NSL

                @pl.when(ck + NSL - 1 < NFULL)
                def _():
                    sdma(ck + NSL - 1, nslot).start()
                    vdma(ck + NSL - 1, nslot).start()
            return accs

        accs = tuple(jnp.zeros((L,), jnp.float32) for _ in range(G))
        accs = lax.fori_loop(0, NJ, jbody, accs)

        rcp.wait()
        vrcp.wait()

        def rstep(i, accs):
            off = i * L
            v16 = vrbuf[pl.ds(off, L)]
            return tuple(
                accs[r] + rbuf[r, pl.ds(off, L)] * v16 for r in range(G))

        accs = lax.fori_loop(0, KREM // L, rstep, accs)
        tcp.wait()
        for i in range(KTAIL // L):
            v16 = vrbuf[pl.ds(KREM + i * L, L)]
            accs = tuple(
                accs[r] + tbuf[r, pl.ds(i * L, L)] * v16 for r in range(G))

        for r in range(G):
            tmat[(g % 2) * G + r] = accs[r]

        if g % 2 == 1:
            outv = jnp.zeros((L,), jnp.float32)
            for c in range(L):
                outv = outv + plsc.load_gather(
                    tmat, [lane, jnp.full((L,), c, jnp.int32)])
            obuf[pl.ds((g // 2) * L, L)] = outv

    pltpu.sync_copy(obuf, o_hbm.at[pl.ds(base, RPW)])


def _sc_call(state, values_flat):
    mesh = plsc.VectorSubcoreMesh(core_axis_name="c", subcore_axis_name="s")
    return pl.kernel(
        _sc_body,
        out_type=jax.ShapeDtypeStruct((B,), jnp.float32),
        mesh=mesh,
        compiler_params=pltpu.CompilerParams(needs_layout_passes=False),
        scratch_types=[
            pltpu.VMEM((NSL, G, KT), jnp.float32),
            pltpu.VMEM((NSL, KT), jnp.float32),
            pltpu.VMEM((G, KREM), jnp.float32),
            pltpu.VMEM((KREM + KTAIL,), jnp.float32),
            pltpu.VMEM((G, KTAIL), jnp.float32),
            pltpu.VMEM((RPW,), jnp.float32),
            pltpu.VMEM((L, L), jnp.float32),
            pltpu.SemaphoreType.DMA((NSL + 2,)),
        ],
    )(state, values_flat)


def kernel(state, values):
    out = _sc_call(state, values.reshape(K))
    return out.reshape(B, 1)
